# baseline (speedup 1.0000x reference)
import functools

import jax
import jax.numpy as jnp
from jax import lax
from jax.experimental import pallas as pl
from jax.experimental.pallas import tpu as pltpu
from jax.experimental.pallas import tpu_sc as plsc

T = 2048
DIM = 1024
HID = 1024
E = 8
BB = 256
NB = 24
NPAD = NB * BB
BT2 = 512

NW = 32


def _router_kernel(x_ref, gw_ref, w01_ref, pos0_ref, pos1_ref, eb_ref,
                   chg_ref, par_ref):
    xb = x_ref[...].astype(jnp.bfloat16)
    gw = gw_ref[...].astype(jnp.bfloat16)
    logits = lax.dot_general(xb, gw, (((1,), (1,)), ((), ())),
                             preferred_element_type=jnp.float32)
    scores = jax.nn.sigmoid(logits)
    lane = lax.broadcasted_iota(jnp.int32, (T, E), 1)
    m1 = jnp.max(scores, axis=1, keepdims=True)
    i1 = jnp.min(jnp.where(scores >= m1, lane, E), axis=1, keepdims=True)
    sel1 = lane == i1
    masked = jnp.where(sel1, -1.0, scores)
    m2 = jnp.max(masked, axis=1, keepdims=True)
    i2 = jnp.min(jnp.where(masked >= m2, lane, E), axis=1, keepdims=True)
    sel2 = lane == i2
    denom = m1 + m2 + 1e-20
    w01_ref[...] = jnp.concatenate([m1 / denom, m2 / denom], axis=1)

    maskE = (sel1 | sel2).astype(jnp.bfloat16)
    r_io = lax.broadcasted_iota(jnp.int32, (T, T), 0)
    c_io = lax.broadcasted_iota(jnp.int32, (T, T), 1)
    tri = (c_io < r_io).astype(jnp.bfloat16)
    rank = lax.dot_general(tri, maskE, (((1,), (0,)), ((), ())),
                           preferred_element_type=jnp.float32)

    maskf = maskE.astype(jnp.float32)
    counts = jnp.sum(maskf, axis=0, keepdims=True)
    nb = jnp.ceil(counts * (1.0 / BB))
    e_r = lax.broadcasted_iota(jnp.int32, (E, E), 0)
    e_c = lax.broadcasted_iota(jnp.int32, (E, E), 1)
    m8 = (e_r < e_c).astype(jnp.float32)
    bs = lax.dot_general(nb, m8, (((1,), (0,)), ((), ())),
                         preferred_element_type=jnp.float32)
    po = bs * float(BB)

    rank_po = rank + po
    pos0 = jnp.sum(jnp.where(sel1, rank_po, 0.0), axis=1, keepdims=True)
    pos1 = jnp.sum(jnp.where(sel2, rank_po, 0.0), axis=1, keepdims=True)
    pos0_ref[...] = pos0.astype(jnp.int32)
    pos1_ref[...] = pos1.astype(jnp.int32)

    ones_t = jnp.ones((T, 1), jnp.float32)
    counts_col = lax.dot_general(maskf, ones_t, (((0,), (0,)), ((), ())))
    nb_col = jnp.ceil(counts_col * (1.0 / BB))
    m8t = (e_c < e_r).astype(jnp.float32)
    bs_col = lax.dot_general(m8t, nb_col, (((1,), (0,)), ((), ())))
    b_io = lax.broadcasted_iota(jnp.int32, (E, NB), 1).astype(jnp.float32)
    e_io = lax.broadcasted_iota(jnp.int32, (E, NB), 0)
    ind = ((b_io >= bs_col) & (e_io >= 1)).astype(jnp.float32)
    ones_e = jnp.ones((1, E), jnp.float32)
    eb = lax.dot_general(ones_e, ind, (((1,), (0,)), ((), ())))
    eb_ref[...] = eb.astype(jnp.int32)

    chg = jnp.concatenate(
        [jnp.zeros((1, 1), jnp.float32),
         (eb[:, 1:] != eb[:, :-1]).astype(jnp.float32)], axis=1)
    b_r = lax.broadcasted_iota(jnp.int32, (NB, NB), 0)
    b_c = lax.broadcasted_iota(jnp.int32, (NB, NB), 1)
    trinb = (b_r <= b_c).astype(jnp.float32)
    cum = lax.dot_general(chg, trinb, (((1,), (0,)), ((), ())))
    par = cum - 2.0 * jnp.floor(cum * 0.5)
    chg_ref[...] = chg.astype(jnp.int32)
    par_ref[...] = par.astype(jnp.int32)


def _router(x, gate_w, interpret=False):
    return pl.pallas_call(
        _router_kernel,
        out_shape=(
            jax.ShapeDtypeStruct((T, 2), jnp.float32),
            jax.ShapeDtypeStruct((T, 1), jnp.int32),
            jax.ShapeDtypeStruct((T, 1), jnp.int32),
            jax.ShapeDtypeStruct((1, NB), jnp.int32),
            jax.ShapeDtypeStruct((1, NB), jnp.int32),
            jax.ShapeDtypeStruct((1, NB), jnp.int32),
        ),
        interpret=interpret,
    )(x, gate_w)


def _sc_dispatch_body(x_hbm, p0_hbm, p1_hbm, xg_hbm, idx_v, rows_v, sem):
    wid = lax.axis_index("s") * 2 + lax.axis_index("c")
    kk = wid // 16
    wi = wid % 16

    @pl.when(kk == 0)
    def _():
        pltpu.sync_copy(p0_hbm.at[wi], idx_v)

    @pl.when(kk == 1)
    def _():
        pltpu.sync_copy(p1_hbm.at[wi], idx_v)

    for c4 in range(4):
        pltpu.sync_copy(x_hbm.at[pl.ds(wi * 128 + c4 * 32, 32)], rows_v)
        pltpu.async_copy(rows_v, xg_hbm.at[idx_v.at[c4]], sem).wait()


def _wcopies(w1_hbm, w3_hbm, w2_hbm, buf, sem, e_idx):
    c1 = pltpu.make_async_copy(w1_hbm.at[e_idx], buf.at[0], sem)
    c3 = pltpu.make_async_copy(w3_hbm.at[e_idx], buf.at[1], sem)
    c2 = pltpu.make_async_copy(w2_hbm.at[e_idx], buf.at[2], sem)
    return c1, c3, c2


def _ffn_kernel(eb_ref, chg_ref, par_ref, xg_ref, w1_hbm, w3_hbm, w2_hbm,
                yg_ref, wa_ref, wb_ref, sema, semb):
    b = pl.program_id(0)
    e = eb_ref[b]
    slot = par_ref[b]
    chg = chg_ref[b]

    @pl.when(b == 0)
    def _():
        for c in _wcopies(w1_hbm, w3_hbm, w2_hbm, wa_ref, sema, e):
            c.start()

    @pl.when(b + 1 < NB)
    def _():
        nxt = jnp.minimum(b + 1, NB - 1)
        en = eb_ref[nxt]

        @pl.when(chg_ref[nxt] == 1)
        def _():
            @pl.when(slot == 0)
            def _():
                for c in _wcopies(w1_hbm, w3_hbm, w2_hbm, wb_ref, semb, en):
                    c.start()

            @pl.when(slot == 1)
            def _():
                for c in _wcopies(w1_hbm, w3_hbm, w2_hbm, wa_ref, sema, en):
                    c.start()

    @pl.when((b == 0) | (chg == 1))
    def _():
        @pl.when(slot == 0)
        def _():
            for c in _wcopies(w1_hbm, w3_hbm, w2_hbm, wa_ref, sema, e):
                c.wait()

        @pl.when(slot == 1)
        def _():
            for c in _wcopies(w1_hbm, w3_hbm, w2_hbm, wb_ref, semb, e):
                c.wait()

    x = xg_ref[...]

    def compute(buf):
        h1 = lax.dot_general(x, buf[0], (((1,), (1,)), ((), ())),
                             preferred_element_type=jnp.float32)
        h3 = lax.dot_general(x, buf[1], (((1,), (1,)), ((), ())),
                             preferred_element_type=jnp.float32)
        h = h1 * jax.nn.sigmoid(h1) * h3
        yg_ref[...] = lax.dot_general(h, buf[2], (((1,), (1,)), ((), ())),
                                      preferred_element_type=jnp.float32)

    @pl.when(slot == 0)
    def _():
        compute(wa_ref)

    @pl.when(slot == 1)
    def _():
        compute(wb_ref)


def _ffn(eb_flat, chg_flat, par_flat, xg, w1, w3, w2, interpret=False):
    grid_spec = pltpu.PrefetchScalarGridSpec(
        num_scalar_prefetch=3,
        grid=(NB,),
        in_specs=[
            pl.BlockSpec((BB, DIM), lambda b, *_: (b, 0)),
            pl.BlockSpec(memory_space=pl.ANY),
            pl.BlockSpec(memory_space=pl.ANY),
            pl.BlockSpec(memory_space=pl.ANY),
        ],
        out_specs=pl.BlockSpec((BB, DIM), lambda b, *_: (b, 0)),
        scratch_shapes=[
            pltpu.VMEM((3, HID, DIM), jnp.float32),
            pltpu.VMEM((3, HID, DIM), jnp.float32),
            pltpu.SemaphoreType.DMA,
            pltpu.SemaphoreType.DMA,
        ],
    )
    return pl.pallas_call(
        _ffn_kernel,
        grid_spec=grid_spec,
        out_shape=jax.ShapeDtypeStruct((NPAD, DIM), jnp.float32),
        interpret=interpret,
    )(eb_flat, chg_flat, par_flat, xg, w1, w3, w2)


def _sc_combine_body(yg_hbm, p0_hbm, p1_hbm, a0_hbm, a1_hbm, idx_v, rows_v, sem):
    wid = lax.axis_index("s") * 2 + lax.axis_index("c")
    base = wid * 64
    pltpu.sync_copy(p0_hbm.at[pl.ds(base, 64)], idx_v)
    pltpu.async_copy(yg_hbm.at[idx_v], rows_v, sem).wait()
    pltpu.sync_copy(rows_v, a0_hbm.at[pl.ds(base, 64)])
    pltpu.sync_copy(p1_hbm.at[pl.ds(base, 64)], idx_v)
    pltpu.async_copy(yg_hbm.at[idx_v], rows_v, sem).wait()
    pltpu.sync_copy(rows_v, a1_hbm.at[pl.ds(base, 64)])


@functools.cache
def _sc_kernels():
    mesh = plsc.VectorSubcoreMesh(core_axis_name="c", subcore_axis_name="s")
    dispatch = pl.kernel(
        _sc_dispatch_body,
        out_type=jax.ShapeDtypeStruct((NPAD, DIM), jnp.float32),
        mesh=mesh,
        scratch_types=[
            pltpu.VMEM((4, 32), jnp.int32),
            pltpu.VMEM((32, DIM), jnp.float32),
            pltpu.SemaphoreType.DMA,
        ],
    )
    combine = pl.kernel(
        _sc_combine_body,
        out_type=(jax.ShapeDtypeStruct((T, DIM), jnp.float32),
                  jax.ShapeDtypeStruct((T, DIM), jnp.float32)),
        mesh=mesh,
        scratch_types=[
            pltpu.VMEM((64,), jnp.int32),
            pltpu.VMEM((64, DIM), jnp.float32),
            pltpu.SemaphoreType.DMA,
        ],
    )
    return dispatch, combine


def _shared_kernel(xb_ref, sw1_ref, sw3_ref, sw2_ref, out_ref):
    x = xb_ref[...]
    h1 = lax.dot_general(x, sw1_ref[...], (((1,), (1,)), ((), ())),
                         preferred_element_type=jnp.float32)
    h3 = lax.dot_general(x, sw3_ref[...], (((1,), (1,)), ((), ())),
                         preferred_element_type=jnp.float32)
    h = h1 * jax.nn.sigmoid(h1) * h3
    out_ref[...] = lax.dot_general(h, sw2_ref[...], (((1,), (1,)), ((), ())),
                                   preferred_element_type=jnp.float32)


def _combine_kernel(sh_ref, a0_ref, a1_ref, w01_ref, out_ref):
    w = w01_ref[...]
    out_ref[...] = (sh_ref[...]
                    + w[:, 0:1] * a0_ref[...]
                    + w[:, 1:2] * a1_ref[...])


def kernel(x, gate_w, w1, w2, w3, sw1, sw2, sw3):
    w01, pos0, pos1, eb, chg, par = _router(x, gate_w)
    p0s = pos0.reshape(16, 4, 32)
    p1s = pos1.reshape(16, 4, 32)
    sc_dispatch, sc_combine = _sc_kernels()
    xg = sc_dispatch(x, p0s, p1s)
    yg = _ffn(eb.reshape(NB), chg.reshape(NB), par.reshape(NB), xg, w1, w3, w2)
    a0, a1 = sc_combine(yg, pos0.reshape(T), pos1.reshape(T))
    shared = pl.pallas_call(
        _shared_kernel,
        grid=(T // BT2,),
        in_specs=[
            pl.BlockSpec((BT2, DIM), lambda i: (i, 0)),
            pl.BlockSpec((HID, DIM), lambda i: (0, 0)),
            pl.BlockSpec((HID, DIM), lambda i: (0, 0)),
            pl.BlockSpec((DIM, HID), lambda i: (0, 0)),
        ],
        out_specs=pl.BlockSpec((BT2, DIM), lambda i: (i, 0)),
        out_shape=jax.ShapeDtypeStruct((T, DIM), jnp.float32),
    )(x, sw1, sw3, sw2)
    out = pl.pallas_call(
        _combine_kernel,
        grid=(T // BT2,),
        in_specs=[
            pl.BlockSpec((BT2, DIM), lambda i: (i, 0)),
            pl.BlockSpec((BT2, DIM), lambda i: (i, 0)),
            pl.BlockSpec((BT2, DIM), lambda i: (i, 0)),
            pl.BlockSpec((BT2, 2), lambda i: (i, 0)),
        ],
        out_specs=pl.BlockSpec((BT2, DIM), lambda i: (i, 0)),
        out_shape=jax.ShapeDtypeStruct((T, DIM), jnp.float32),
    )(shared, a0, a1, w01)
    return out


# speedup vs baseline: 1.1046x; 1.1046x over previous
import functools

import jax
import jax.numpy as jnp
from jax import lax
from jax.experimental import pallas as pl
from jax.experimental.pallas import tpu as pltpu
from jax.experimental.pallas import tpu_sc as plsc

T = 2048
DIM = 1024
HID = 1024
E = 8
BB = 256
NB = 24
NPAD = NB * BB
BT2 = 512

NW = 32


def _router_kernel(x_ref, gw_ref, w01_ref, pos0_ref, pos1_ref, eb_ref,
                   chg_ref, par_ref, nxt_ref):
    xb = x_ref[...].astype(jnp.bfloat16)
    gw = gw_ref[...].astype(jnp.bfloat16)
    logits = lax.dot_general(xb, gw, (((1,), (1,)), ((), ())),
                             preferred_element_type=jnp.float32)
    scores = jax.nn.sigmoid(logits)
    lane = lax.broadcasted_iota(jnp.int32, (T, E), 1)
    m1 = jnp.max(scores, axis=1, keepdims=True)
    i1 = jnp.min(jnp.where(scores >= m1, lane, E), axis=1, keepdims=True)
    sel1 = lane == i1
    masked = jnp.where(sel1, -1.0, scores)
    m2 = jnp.max(masked, axis=1, keepdims=True)
    i2 = jnp.min(jnp.where(masked >= m2, lane, E), axis=1, keepdims=True)
    sel2 = lane == i2
    denom = m1 + m2 + 1e-20
    w01_ref[...] = jnp.concatenate([m1 / denom, m2 / denom], axis=1)

    maskE = (sel1 | sel2).astype(jnp.bfloat16)
    r_io = lax.broadcasted_iota(jnp.int32, (T, T), 0)
    c_io = lax.broadcasted_iota(jnp.int32, (T, T), 1)
    tri = (c_io < r_io).astype(jnp.bfloat16)
    rank = lax.dot_general(tri, maskE, (((1,), (0,)), ((), ())),
                           preferred_element_type=jnp.float32)

    maskf = maskE.astype(jnp.float32)
    counts = jnp.sum(maskf, axis=0, keepdims=True)
    nb = jnp.ceil(counts * (1.0 / BB))
    e_r = lax.broadcasted_iota(jnp.int32, (E, E), 0)
    e_c = lax.broadcasted_iota(jnp.int32, (E, E), 1)
    m8 = (e_r < e_c).astype(jnp.float32)
    bs = lax.dot_general(nb, m8, (((1,), (0,)), ((), ())),
                         preferred_element_type=jnp.float32)
    po = bs * float(BB)

    rank_po = rank + po
    pos0 = jnp.sum(jnp.where(sel1, rank_po, 0.0), axis=1, keepdims=True)
    pos1 = jnp.sum(jnp.where(sel2, rank_po, 0.0), axis=1, keepdims=True)
    pos0_ref[...] = pos0.astype(jnp.int32)
    pos1_ref[...] = pos1.astype(jnp.int32)

    ones_t = jnp.ones((T, 1), jnp.float32)
    counts_col = lax.dot_general(maskf, ones_t, (((0,), (0,)), ((), ())))
    nb_col = jnp.ceil(counts_col * (1.0 / BB))
    m8t = (e_c < e_r).astype(jnp.float32)
    bs_col = lax.dot_general(m8t, nb_col, (((1,), (0,)), ((), ())))
    b_io = lax.broadcasted_iota(jnp.int32, (E, NB), 1).astype(jnp.float32)
    e_io = lax.broadcasted_iota(jnp.int32, (E, NB), 0)
    ind = ((b_io >= bs_col) & (e_io >= 1)).astype(jnp.float32)
    ones_e = jnp.ones((1, E), jnp.float32)
    eb = lax.dot_general(ones_e, ind, (((1,), (0,)), ((), ())))
    eb_ref[...] = eb.astype(jnp.int32)

    chg = jnp.concatenate(
        [jnp.zeros((1, 1), jnp.float32),
         (eb[:, 1:] != eb[:, :-1]).astype(jnp.float32)], axis=1)
    b_r = lax.broadcasted_iota(jnp.int32, (NB, NB), 0)
    b_c = lax.broadcasted_iota(jnp.int32, (NB, NB), 1)
    trinb = (b_r <= b_c).astype(jnp.float32)
    cum = lax.dot_general(chg, trinb, (((1,), (0,)), ((), ())))
    par = cum - 2.0 * jnp.floor(cum * 0.5)
    chg_ref[...] = chg.astype(jnp.int32)
    par_ref[...] = par.astype(jnp.int32)

    ones_ecol = jnp.ones((E, 1), jnp.float32)
    ebc = lax.dot_general(ind, ones_ecol, (((0,), (0,)), ((), ())))
    chg_col = jnp.concatenate(
        [jnp.zeros((1, 1), jnp.float32),
         (ebc[1:] != ebc[:-1]).astype(jnp.float32)], axis=0)
    cand = jnp.where((b_r > b_c) & (chg_col > 0.5), b_r, NB)
    nxt_ref[...] = jnp.min(cand, axis=0, keepdims=True).astype(jnp.int32)


def _router(x, gate_w, interpret=False):
    return pl.pallas_call(
        _router_kernel,
        out_shape=(
            jax.ShapeDtypeStruct((T, 2), jnp.float32),
            jax.ShapeDtypeStruct((T, 1), jnp.int32),
            jax.ShapeDtypeStruct((T, 1), jnp.int32),
            jax.ShapeDtypeStruct((1, NB), jnp.int32),
            jax.ShapeDtypeStruct((1, NB), jnp.int32),
            jax.ShapeDtypeStruct((1, NB), jnp.int32),
            jax.ShapeDtypeStruct((1, NB), jnp.int32),
        ),
        interpret=interpret,
    )(x, gate_w)


def _sc_dispatch_body(x_hbm, p0_hbm, p1_hbm, xg_hbm, idx_v, rows_v, sem):
    wid = lax.axis_index("s") * 2 + lax.axis_index("c")
    kk = wid // 16
    wi = wid % 16

    @pl.when(kk == 0)
    def _():
        pltpu.sync_copy(p0_hbm.at[wi], idx_v)

    @pl.when(kk == 1)
    def _():
        pltpu.sync_copy(p1_hbm.at[wi], idx_v)

    for c4 in range(4):
        pltpu.sync_copy(x_hbm.at[pl.ds(wi * 128 + c4 * 32, 32)], rows_v)
        pltpu.async_copy(rows_v, xg_hbm.at[idx_v.at[c4]], sem).wait()


def _wcopies(w1_hbm, w3_hbm, w2_hbm, buf, sem, e_idx):
    c1 = pltpu.make_async_copy(w1_hbm.at[e_idx], buf.at[0], sem)
    c3 = pltpu.make_async_copy(w3_hbm.at[e_idx], buf.at[1], sem)
    c2 = pltpu.make_async_copy(w2_hbm.at[e_idx], buf.at[2], sem)
    return c1, c3, c2


def _ffn_kernel(eb_ref, chg_ref, par_ref, nxt_ref, xg_ref,
                w1_hbm, w3_hbm, w2_hbm, yg_ref, wa_ref, wb_ref,
                sema, semb, iss_ref):
    b = pl.program_id(0)
    e = eb_ref[b]
    slot = par_ref[b]
    chg = chg_ref[b]

    @pl.when(b == 0)
    def _():
        iss_ref[0] = 0
        for c in _wcopies(w1_hbm, w3_hbm, w2_hbm, wa_ref, sema, e):
            c.start()

    @pl.when((b == 0) | (chg == 1))
    def _():
        @pl.when(slot == 0)
        def _():
            for c in _wcopies(w1_hbm, w3_hbm, w2_hbm, wa_ref, sema, e):
                c.wait()

        @pl.when(slot == 1)
        def _():
            for c in _wcopies(w1_hbm, w3_hbm, w2_hbm, wb_ref, semb, e):
                c.wait()

    n = nxt_ref[b]

    @pl.when((n < NB) & (iss_ref[0] < n))
    def _():
        nn = jnp.minimum(n, NB - 1)
        en = eb_ref[nn]
        iss_ref[0] = n

        @pl.when(slot == 0)
        def _():
            for c in _wcopies(w1_hbm, w3_hbm, w2_hbm, wb_ref, semb, en):
                c.start()

        @pl.when(slot == 1)
        def _():
            for c in _wcopies(w1_hbm, w3_hbm, w2_hbm, wa_ref, sema, en):
                c.start()

    x = xg_ref[...]

    def compute(buf):
        h1 = lax.dot_general(x, buf[0], (((1,), (1,)), ((), ())),
                             preferred_element_type=jnp.float32)
        h3 = lax.dot_general(x, buf[1], (((1,), (1,)), ((), ())),
                             preferred_element_type=jnp.float32)
        h = h1 * jax.nn.sigmoid(h1) * h3
        yg_ref[...] = lax.dot_general(h, buf[2], (((1,), (1,)), ((), ())),
                                      preferred_element_type=jnp.float32)

    @pl.when(slot == 0)
    def _():
        compute(wa_ref)

    @pl.when(slot == 1)
    def _():
        compute(wb_ref)


def _ffn(eb_flat, chg_flat, par_flat, nxt_flat, xg, w1, w3, w2,
         interpret=False):
    grid_spec = pltpu.PrefetchScalarGridSpec(
        num_scalar_prefetch=4,
        grid=(NB,),
        in_specs=[
            pl.BlockSpec((BB, DIM), lambda b, *_: (b, 0)),
            pl.BlockSpec(memory_space=pl.ANY),
            pl.BlockSpec(memory_space=pl.ANY),
            pl.BlockSpec(memory_space=pl.ANY),
        ],
        out_specs=pl.BlockSpec((BB, DIM), lambda b, *_: (b, 0)),
        scratch_shapes=[
            pltpu.VMEM((3, HID, DIM), jnp.float32),
            pltpu.VMEM((3, HID, DIM), jnp.float32),
            pltpu.SemaphoreType.DMA,
            pltpu.SemaphoreType.DMA,
            pltpu.SMEM((1,), jnp.int32),
        ],
    )
    return pl.pallas_call(
        _ffn_kernel,
        grid_spec=grid_spec,
        out_shape=jax.ShapeDtypeStruct((NPAD, DIM), jnp.float32),
        interpret=interpret,
    )(eb_flat, chg_flat, par_flat, nxt_flat, xg, w1, w3, w2)


def _sc_combine_body(yg_hbm, p0_hbm, p1_hbm, a0_hbm, a1_hbm, idx_v, rows_v, sem):
    wid = lax.axis_index("s") * 2 + lax.axis_index("c")
    base = wid * 64
    pltpu.sync_copy(p0_hbm.at[pl.ds(base, 64)], idx_v)
    pltpu.async_copy(yg_hbm.at[idx_v], rows_v, sem).wait()
    pltpu.sync_copy(rows_v, a0_hbm.at[pl.ds(base, 64)])
    pltpu.sync_copy(p1_hbm.at[pl.ds(base, 64)], idx_v)
    pltpu.async_copy(yg_hbm.at[idx_v], rows_v, sem).wait()
    pltpu.sync_copy(rows_v, a1_hbm.at[pl.ds(base, 64)])


@functools.cache
def _sc_kernels():
    mesh = plsc.VectorSubcoreMesh(core_axis_name="c", subcore_axis_name="s")
    dispatch = pl.kernel(
        _sc_dispatch_body,
        out_type=jax.ShapeDtypeStruct((NPAD, DIM), jnp.float32),
        mesh=mesh,
        scratch_types=[
            pltpu.VMEM((4, 32), jnp.int32),
            pltpu.VMEM((32, DIM), jnp.float32),
            pltpu.SemaphoreType.DMA,
        ],
    )
    combine = pl.kernel(
        _sc_combine_body,
        out_type=(jax.ShapeDtypeStruct((T, DIM), jnp.float32),
                  jax.ShapeDtypeStruct((T, DIM), jnp.float32)),
        mesh=mesh,
        scratch_types=[
            pltpu.VMEM((64,), jnp.int32),
            pltpu.VMEM((64, DIM), jnp.float32),
            pltpu.SemaphoreType.DMA,
        ],
    )
    return dispatch, combine


def _shared_kernel(xb_ref, sw1_ref, sw3_ref, sw2_ref, out_ref):
    x = xb_ref[...]
    h1 = lax.dot_general(x, sw1_ref[...], (((1,), (1,)), ((), ())),
                         preferred_element_type=jnp.float32)
    h3 = lax.dot_general(x, sw3_ref[...], (((1,), (1,)), ((), ())),
                         preferred_element_type=jnp.float32)
    h = h1 * jax.nn.sigmoid(h1) * h3
    out_ref[...] = lax.dot_general(h, sw2_ref[...], (((1,), (1,)), ((), ())),
                                   preferred_element_type=jnp.float32)


def _combine_kernel(sh_ref, a0_ref, a1_ref, w01_ref, out_ref):
    w = w01_ref[...]
    out_ref[...] = (sh_ref[...]
                    + w[:, 0:1] * a0_ref[...]
                    + w[:, 1:2] * a1_ref[...])


def kernel(x, gate_w, w1, w2, w3, sw1, sw2, sw3):
    w01, pos0, pos1, eb, chg, par, nxt = _router(x, gate_w)
    p0s = pos0.reshape(16, 4, 32)
    p1s = pos1.reshape(16, 4, 32)
    sc_dispatch, sc_combine = _sc_kernels()
    xg = sc_dispatch(x, p0s, p1s)
    yg = _ffn(eb.reshape(NB), chg.reshape(NB), par.reshape(NB),
              nxt.reshape(NB), xg, w1, w3, w2)
    a0, a1 = sc_combine(yg, pos0.reshape(T), pos1.reshape(T))
    shared = pl.pallas_call(
        _shared_kernel,
        grid=(T // BT2,),
        in_specs=[
            pl.BlockSpec((BT2, DIM), lambda i: (i, 0)),
            pl.BlockSpec((HID, DIM), lambda i: (0, 0)),
            pl.BlockSpec((HID, DIM), lambda i: (0, 0)),
            pl.BlockSpec((DIM, HID), lambda i: (0, 0)),
        ],
        out_specs=pl.BlockSpec((BT2, DIM), lambda i: (i, 0)),
        out_shape=jax.ShapeDtypeStruct((T, DIM), jnp.float32),
    )(x, sw1, sw3, sw2)
    out = pl.pallas_call(
        _combine_kernel,
        grid=(T // BT2,),
        in_specs=[
            pl.BlockSpec((BT2, DIM), lambda i: (i, 0)),
            pl.BlockSpec((BT2, DIM), lambda i: (i, 0)),
            pl.BlockSpec((BT2, DIM), lambda i: (i, 0)),
            pl.BlockSpec((BT2, 2), lambda i: (i, 0)),
        ],
        out_specs=pl.BlockSpec((BT2, DIM), lambda i: (i, 0)),
        out_shape=jax.ShapeDtypeStruct((T, DIM), jnp.float32),
    )(shared, a0, a1, w01)
    return out
